# baseline (speedup 1.0000x reference)
import functools

import jax
import jax.numpy as jnp
import numpy as np
from jax import lax
from jax.experimental import pallas as pl
from jax.experimental.pallas import tpu as pltpu
from jax.experimental.pallas import tpu_sc as plsc

N = 10000
E = 320000
W = 128
NBLK = 5
NCORES = 2
NTILES = 16
NPAD = 10240
ROWS_PER_TILE = NPAD // NTILES
KCHUNKS = 2
EC = E // KCHUNKS
EDGES_PER_TILE = EC // (NCORES * NTILES)
CHUNK = 40
NCHUNK = EDGES_PER_TILE // CHUNK

_M = np.concatenate([
    np.arange(128),
    128 + np.repeat(np.arange(64), 3),
    192 + np.repeat(np.arange(32), 5),
])
_MFULL = np.concatenate([_M, 224 + _M, np.arange(448, 576)])


def _node_kernel(ni_ref, ne_ref, ln_w_ref, ln_b_ref, o3_w0_ref, o3_b0_ref,
                 w1rep_ref, w2rep_ref, W1_ref, b1_ref, h_ref, ne_out_ref):
    ni = ni_ref[...]
    ne = ne_ref[...]
    mu = jnp.mean(ni, axis=-1, keepdims=True)
    var = jnp.mean((ni - mu) ** 2, axis=-1, keepdims=True)
    ns = (ni - mu) * jax.lax.rsqrt(var + 1e-5) * ln_w_ref[...] + ln_b_ref[...]
    s = ne[:, :128]
    v1 = ne[:, 128:320]
    v2 = ne[:, 320:480]
    mu_s = jnp.mean(s, axis=-1, keepdims=True)
    var_s = jnp.mean((s - mu_s) ** 2, axis=-1, keepdims=True)
    s_n = (s - mu_s) * jax.lax.rsqrt(var_s + 1e-5) * o3_w0_ref[...] + o3_b0_ref[...]
    inv_rms1 = jax.lax.rsqrt(jnp.sum(v1 * v1, axis=-1, keepdims=True) / 64.0 + 1e-5)
    v1_n = v1 * inv_rms1 * w1rep_ref[...]
    inv_rms2 = jax.lax.rsqrt(jnp.sum(v2 * v2, axis=-1, keepdims=True) / 32.0 + 1e-5)
    v2_n = v2 * inv_rms2 * w2rep_ref[...]
    ne_out_ref[...] = jnp.concatenate([s_n, v1_n, v2_n], axis=-1)
    hpre = ns @ W1_ref[...] + b1_ref[...]
    h_ref[...] = hpre * jax.nn.sigmoid(hpre)


def _edge_kernel(hg_ref, neg_ref, rsh_ref, rbf_ref, fcut_ref,
                 W2e_ref, b2e_ref, rbf_We_ref, rbf_be_ref,
                 m0_ref, m1_ref, m2_ref, m3_ref, m4_ref):
    hg = hg_ref[...]
    fwe = (rbf_ref[...] @ rbf_We_ref[...] + rbf_be_ref[...]) * fcut_ref[...]
    foe = (hg @ W2e_ref[...] + b2e_ref[...]) * fwe
    msg_eq = neg_ref[...] * foe[:, :480] + rsh_ref[...] * foe[:, 480:960]
    m0_ref[...] = foe[:, 960:1088]
    m1_ref[...] = msg_eq[:, :128]
    m2_ref[...] = msg_eq[:, 128:256]
    m3_ref[...] = msg_eq[:, 256:384]
    bs = msg_eq.shape[0]
    m4_ref[...] = jnp.concatenate(
        [msg_eq[:, 384:480], jnp.zeros((bs, 32), jnp.float32)], axis=-1)


def _sc_phase(cid, tid, m_hbm, init_hbm, out_hbm, center_hbm, acc, buf, idxbuf):
    r0 = tid * ROWS_PER_TILE
    pltpu.sync_copy(init_hbm.at[pl.ds(cid * NPAD + r0, ROWS_PER_TILE)],
                    acc.at[pl.ds(r0, ROWS_PER_TILE)])
    plsc.subcore_barrier()

    ebase = cid * (EC // NCORES) + tid * EDGES_PER_TILE

    def body(j, carry):
        e0 = ebase + j * CHUNK
        pltpu.sync_copy(m_hbm.at[pl.ds(e0, CHUNK)], buf)
        pltpu.sync_copy(center_hbm.at[pl.ds(e0, CHUNK)], idxbuf)
        pltpu.sync_copy(buf, acc.at[idxbuf], add=True)
        return carry

    lax.fori_loop(0, NCHUNK, body, 0)
    plsc.subcore_barrier()
    pltpu.sync_copy(acc.at[pl.ds(r0, ROWS_PER_TILE)],
                    out_hbm.at[pl.ds(cid * NPAD + r0, ROWS_PER_TILE)])
    plsc.subcore_barrier()


def _make_scatter_kernel():
    mesh = plsc.VectorSubcoreMesh(core_axis_name="c", subcore_axis_name="s")
    out_type = [jax.ShapeDtypeStruct((NCORES * NPAD, W), jnp.float32)
                for _ in range(NBLK)]
    scratch_types = [
        pltpu.VMEM_SHARED((NPAD, W), jnp.float32),
        pltpu.VMEM((CHUNK, W), jnp.float32),
        pltpu.VMEM((CHUNK,), jnp.int32),
    ]

    @functools.partial(pl.kernel, mesh=mesh, out_type=out_type,
                       scratch_types=scratch_types)
    def scatter_kernel(m0, m1, m2, m3, m4, center, i0, i1, i2, i3, i4,
                       o0, o1, o2, o3, o4, acc, buf, idxbuf):
        cid = lax.axis_index("c")
        tid = lax.axis_index("s")
        for m, i, o in ((m0, i0, o0), (m1, i1, o1), (m2, i2, o2),
                        (m3, i3, o3), (m4, i4, o4)):
            _sc_phase(cid, tid, m, i, o, center, acc, buf, idxbuf)

    return scatter_kernel


def kernel(node_invariant, node_equivariant, rbf, fcut, rsh, edge_index, ln_w, ln_b, o3_w0, o3_b0, o3_w1, o3_w2, W1, b1, W2, b2, rbf_W, rbf_b):
    w1rep = jnp.repeat(o3_w1, 3)
    w2rep = jnp.repeat(o3_w2, 5)
    mfull = jnp.asarray(_MFULL, dtype=jnp.int32)
    W2e = W2[:, mfull]
    b2e = b2[mfull]
    rbf_We = rbf_W[:, mfull]
    rbf_be = rbf_b[mfull]

    nb = 10
    bs = N // nb
    h, ne_norm = pl.pallas_call(
        _node_kernel,
        grid=(nb,),
        in_specs=[
            pl.BlockSpec((bs, 128), lambda i: (i, 0)),
            pl.BlockSpec((bs, 480), lambda i: (i, 0)),
            pl.BlockSpec((128,), lambda i: (0,)),
            pl.BlockSpec((128,), lambda i: (0,)),
            pl.BlockSpec((128,), lambda i: (0,)),
            pl.BlockSpec((128,), lambda i: (0,)),
            pl.BlockSpec((192,), lambda i: (0,)),
            pl.BlockSpec((160,), lambda i: (0,)),
            pl.BlockSpec((128, 128), lambda i: (0, 0)),
            pl.BlockSpec((128,), lambda i: (0,)),
        ],
        out_specs=[
            pl.BlockSpec((bs, 128), lambda i: (i, 0)),
            pl.BlockSpec((bs, 480), lambda i: (i, 0)),
        ],
        out_shape=[
            jax.ShapeDtypeStruct((N, 128), jnp.float32),
            jax.ShapeDtypeStruct((N, 480), jnp.float32),
        ],
    )(node_invariant, node_equivariant, ln_w, ln_b, o3_w0, o3_b0,
      w1rep, w2rep, W1, b1)

    center = edge_index[0]
    neigh = edge_index[1]

    eb = 160
    ebs = EC // eb
    scatter = _make_scatter_kernel()
    parts = [jnp.zeros((NCORES * NPAD, W), jnp.float32) for _ in range(NBLK)]
    for k in range(KCHUNKS):
        sl = slice(k * EC, (k + 1) * EC)
        neigh_k = neigh[sl]
        h_g = h[neigh_k]
        ne_g = ne_norm[neigh_k]
        msgs = pl.pallas_call(
            _edge_kernel,
            grid=(eb,),
            in_specs=[
                pl.BlockSpec((ebs, 128), lambda i: (i, 0)),
                pl.BlockSpec((ebs, 480), lambda i: (i, 0)),
                pl.BlockSpec((ebs, 480), lambda i: (i, 0)),
                pl.BlockSpec((ebs, 20), lambda i: (i, 0)),
                pl.BlockSpec((ebs, 1), lambda i: (i, 0)),
                pl.BlockSpec((128, 1088), lambda i: (0, 0)),
                pl.BlockSpec((1088,), lambda i: (0,)),
                pl.BlockSpec((20, 1088), lambda i: (0, 0)),
                pl.BlockSpec((1088,), lambda i: (0,)),
            ],
            out_specs=[
                pl.BlockSpec((ebs, W), lambda i: (i, 0)) for _ in range(NBLK)
            ],
            out_shape=[jax.ShapeDtypeStruct((EC, W), jnp.float32)
                       for _ in range(NBLK)],
        )(h_g, ne_g, rsh[sl], rbf[sl], fcut[sl], W2e, b2e, rbf_We, rbf_be)

        parts = list(scatter(*msgs, center[sl], *parts))

    o0, o1, o2, o3, o4 = parts
    new_inv = node_invariant + o0[:N] + o0[NPAD:NPAD + N]
    eqs = [o[:N] + o[NPAD:NPAD + N] for o in (o1, o2, o3, o4)]
    new_eq = node_equivariant + jnp.concatenate(
        [eqs[0], eqs[1], eqs[2], eqs[3][:, :96]], axis=1)
    return new_inv, new_eq


# speedup vs baseline: 1.1156x; 1.1156x over previous
import functools

import jax
import jax.numpy as jnp
import numpy as np
from jax import lax
from jax.experimental import pallas as pl
from jax.experimental.pallas import tpu as pltpu
from jax.experimental.pallas import tpu_sc as plsc

N = 10000
E = 320000
W = 128
NBLK = 5
NCORES = 2
NTILES = 16
NPAD = 10240
ROWS_PER_TILE = NPAD // NTILES
KCHUNKS = 1
EC = E // KCHUNKS
EDGES_PER_TILE = EC // (NCORES * NTILES)
CHUNK = 128
NCHUNK = EDGES_PER_TILE // CHUNK
TAIL = EDGES_PER_TILE - NCHUNK * CHUNK

_M = np.concatenate([
    np.arange(128),
    128 + np.repeat(np.arange(64), 3),
    192 + np.repeat(np.arange(32), 5),
])
_MFULL = np.concatenate([_M, 224 + _M, np.arange(448, 576)])


def _node_kernel(ni_ref, ne_ref, ln_w_ref, ln_b_ref, o3_w0_ref, o3_b0_ref,
                 w1rep_ref, w2rep_ref, W1_ref, b1_ref, h_ref, ne_out_ref):
    ni = ni_ref[...]
    ne = ne_ref[...]
    mu = jnp.mean(ni, axis=-1, keepdims=True)
    var = jnp.mean((ni - mu) ** 2, axis=-1, keepdims=True)
    ns = (ni - mu) * jax.lax.rsqrt(var + 1e-5) * ln_w_ref[...] + ln_b_ref[...]
    s = ne[:, :128]
    v1 = ne[:, 128:320]
    v2 = ne[:, 320:480]
    mu_s = jnp.mean(s, axis=-1, keepdims=True)
    var_s = jnp.mean((s - mu_s) ** 2, axis=-1, keepdims=True)
    s_n = (s - mu_s) * jax.lax.rsqrt(var_s + 1e-5) * o3_w0_ref[...] + o3_b0_ref[...]
    inv_rms1 = jax.lax.rsqrt(jnp.sum(v1 * v1, axis=-1, keepdims=True) / 64.0 + 1e-5)
    v1_n = v1 * inv_rms1 * w1rep_ref[...]
    inv_rms2 = jax.lax.rsqrt(jnp.sum(v2 * v2, axis=-1, keepdims=True) / 32.0 + 1e-5)
    v2_n = v2 * inv_rms2 * w2rep_ref[...]
    ne_out_ref[...] = jnp.concatenate([s_n, v1_n, v2_n], axis=-1)
    hpre = ns @ W1_ref[...] + b1_ref[...]
    h_ref[...] = hpre * jax.nn.sigmoid(hpre)


def _edge_kernel(hg_ref, neg_ref, rsh_ref, rbf_ref, fcut_ref,
                 W2e_ref, b2e_ref, rbf_We_ref, rbf_be_ref,
                 m0_ref, m1_ref, m2_ref, m3_ref, m4_ref):
    hg = hg_ref[...]
    fwe = (rbf_ref[...] @ rbf_We_ref[...] + rbf_be_ref[...]) * fcut_ref[...]
    foe = (hg @ W2e_ref[...] + b2e_ref[...]) * fwe
    msg_eq = neg_ref[...] * foe[:, :480] + rsh_ref[...] * foe[:, 480:960]
    m0_ref[...] = foe[:, 960:1088]
    m1_ref[...] = msg_eq[:, :128]
    m2_ref[...] = msg_eq[:, 128:256]
    m3_ref[...] = msg_eq[:, 256:384]
    bs = msg_eq.shape[0]
    m4_ref[...] = jnp.concatenate(
        [msg_eq[:, 384:480], jnp.zeros((bs, 32), jnp.float32)], axis=-1)


def _sc_phase(cid, tid, m_hbm, init_hbm, out_hbm, center_hbm,
              acc, buf, idxbuf, buf_t, idx_t):
    r0 = tid * ROWS_PER_TILE
    pltpu.sync_copy(init_hbm.at[pl.ds(cid * NPAD + r0, ROWS_PER_TILE)],
                    acc.at[pl.ds(r0, ROWS_PER_TILE)])
    plsc.subcore_barrier()

    ebase = cid * (EC // NCORES) + tid * EDGES_PER_TILE

    def body(j, carry):
        e0 = ebase + j * CHUNK
        pltpu.sync_copy(m_hbm.at[pl.ds(e0, CHUNK)], buf)
        pltpu.sync_copy(center_hbm.at[pl.ds(e0, CHUNK)], idxbuf)
        pltpu.sync_copy(buf, acc.at[idxbuf], add=True)
        return carry

    lax.fori_loop(0, NCHUNK, body, 0)
    e0t = ebase + NCHUNK * CHUNK
    pltpu.sync_copy(m_hbm.at[pl.ds(e0t, TAIL)], buf_t)
    pltpu.sync_copy(center_hbm.at[pl.ds(e0t, TAIL)], idx_t)
    pltpu.sync_copy(buf_t, acc.at[idx_t], add=True)
    plsc.subcore_barrier()
    pltpu.sync_copy(acc.at[pl.ds(r0, ROWS_PER_TILE)],
                    out_hbm.at[pl.ds(cid * NPAD + r0, ROWS_PER_TILE)])
    plsc.subcore_barrier()


def _make_scatter_kernel():
    mesh = plsc.VectorSubcoreMesh(core_axis_name="c", subcore_axis_name="s")
    out_type = [jax.ShapeDtypeStruct((NCORES * NPAD, W), jnp.float32)
                for _ in range(NBLK)]
    scratch_types = [
        pltpu.VMEM_SHARED((NPAD, W), jnp.float32),
        pltpu.VMEM((CHUNK, W), jnp.float32),
        pltpu.VMEM((CHUNK,), jnp.int32),
        pltpu.VMEM((TAIL, W), jnp.float32),
        pltpu.VMEM((TAIL,), jnp.int32),
    ]

    @functools.partial(pl.kernel, mesh=mesh, out_type=out_type,
                       scratch_types=scratch_types)
    def scatter_kernel(m0, m1, m2, m3, m4, center, i0, i1, i2, i3, i4,
                       o0, o1, o2, o3, o4, acc, buf, idxbuf, buf_t, idx_t):
        cid = lax.axis_index("c")
        tid = lax.axis_index("s")
        for m, i, o in ((m0, i0, o0), (m1, i1, o1), (m2, i2, o2),
                        (m3, i3, o3), (m4, i4, o4)):
            _sc_phase(cid, tid, m, i, o, center, acc, buf, idxbuf, buf_t, idx_t)

    return scatter_kernel


def kernel(node_invariant, node_equivariant, rbf, fcut, rsh, edge_index, ln_w, ln_b, o3_w0, o3_b0, o3_w1, o3_w2, W1, b1, W2, b2, rbf_W, rbf_b):
    w1rep = jnp.repeat(o3_w1, 3)
    w2rep = jnp.repeat(o3_w2, 5)
    mfull = jnp.asarray(_MFULL, dtype=jnp.int32)
    W2e = W2[:, mfull]
    b2e = b2[mfull]
    rbf_We = rbf_W[:, mfull]
    rbf_be = rbf_b[mfull]

    nb = 10
    bs = N // nb
    h, ne_norm = pl.pallas_call(
        _node_kernel,
        grid=(nb,),
        in_specs=[
            pl.BlockSpec((bs, 128), lambda i: (i, 0)),
            pl.BlockSpec((bs, 480), lambda i: (i, 0)),
            pl.BlockSpec((128,), lambda i: (0,)),
            pl.BlockSpec((128,), lambda i: (0,)),
            pl.BlockSpec((128,), lambda i: (0,)),
            pl.BlockSpec((128,), lambda i: (0,)),
            pl.BlockSpec((192,), lambda i: (0,)),
            pl.BlockSpec((160,), lambda i: (0,)),
            pl.BlockSpec((128, 128), lambda i: (0, 0)),
            pl.BlockSpec((128,), lambda i: (0,)),
        ],
        out_specs=[
            pl.BlockSpec((bs, 128), lambda i: (i, 0)),
            pl.BlockSpec((bs, 480), lambda i: (i, 0)),
        ],
        out_shape=[
            jax.ShapeDtypeStruct((N, 128), jnp.float32),
            jax.ShapeDtypeStruct((N, 480), jnp.float32),
        ],
    )(node_invariant, node_equivariant, ln_w, ln_b, o3_w0, o3_b0,
      w1rep, w2rep, W1, b1)

    center = edge_index[0]
    neigh = edge_index[1]

    eb = 160
    ebs = EC // eb
    scatter = _make_scatter_kernel()
    parts = [jnp.zeros((NCORES * NPAD, W), jnp.float32) for _ in range(NBLK)]
    for k in range(KCHUNKS):
        sl = slice(k * EC, (k + 1) * EC)
        neigh_k = neigh[sl]
        h_g = h[neigh_k]
        ne_g = ne_norm[neigh_k]
        msgs = pl.pallas_call(
            _edge_kernel,
            grid=(eb,),
            in_specs=[
                pl.BlockSpec((ebs, 128), lambda i: (i, 0)),
                pl.BlockSpec((ebs, 480), lambda i: (i, 0)),
                pl.BlockSpec((ebs, 480), lambda i: (i, 0)),
                pl.BlockSpec((ebs, 20), lambda i: (i, 0)),
                pl.BlockSpec((ebs, 1), lambda i: (i, 0)),
                pl.BlockSpec((128, 1088), lambda i: (0, 0)),
                pl.BlockSpec((1088,), lambda i: (0,)),
                pl.BlockSpec((20, 1088), lambda i: (0, 0)),
                pl.BlockSpec((1088,), lambda i: (0,)),
            ],
            out_specs=[
                pl.BlockSpec((ebs, W), lambda i: (i, 0)) for _ in range(NBLK)
            ],
            out_shape=[jax.ShapeDtypeStruct((EC, W), jnp.float32)
                       for _ in range(NBLK)],
        )(h_g, ne_g, rsh[sl], rbf[sl], fcut[sl], W2e, b2e, rbf_We, rbf_be)

        parts = list(scatter(*msgs, center[sl], *parts))

    o0, o1, o2, o3, o4 = parts
    new_inv = node_invariant + o0[:N] + o0[NPAD:NPAD + N]
    eqs = [o[:N] + o[NPAD:NPAD + N] for o in (o1, o2, o3, o4)]
    new_eq = node_equivariant + jnp.concatenate(
        [eqs[0], eqs[1], eqs[2], eqs[3][:, :96]], axis=1)
    return new_inv, new_eq


# speedup vs baseline: 1.2650x; 1.1339x over previous
import functools

import jax
import jax.numpy as jnp
import numpy as np
from jax import lax
from jax.experimental import pallas as pl
from jax.experimental.pallas import tpu as pltpu
from jax.experimental.pallas import tpu_sc as plsc

N = 10000
E = 320000
W = 128
NBLK = 5
NCORES = 2
NTILES = 16
NPAD = 10240
ROWS_PER_TILE = NPAD // NTILES
KCHUNKS = 1
EC = E // KCHUNKS
EDGES_PER_TILE = EC // (NCORES * NTILES)
CHUNK = 128
NCHUNK = EDGES_PER_TILE // CHUNK
TAIL = EDGES_PER_TILE - NCHUNK * CHUNK

_M = np.concatenate([
    np.arange(128),
    128 + np.repeat(np.arange(64), 3),
    192 + np.repeat(np.arange(32), 5),
])
_MFULL = np.concatenate([_M, 224 + _M, np.arange(448, 576)])


def _node_kernel(ni_ref, ne_ref, ln_w_ref, ln_b_ref, o3_w0_ref, o3_b0_ref,
                 w1rep_ref, w2rep_ref, W1_ref, b1_ref, h_ref, ne_out_ref):
    ni = ni_ref[...]
    ne = ne_ref[...]
    mu = jnp.mean(ni, axis=-1, keepdims=True)
    var = jnp.mean((ni - mu) ** 2, axis=-1, keepdims=True)
    ns = (ni - mu) * jax.lax.rsqrt(var + 1e-5) * ln_w_ref[...] + ln_b_ref[...]
    s = ne[:, :128]
    v1 = ne[:, 128:320]
    v2 = ne[:, 320:480]
    mu_s = jnp.mean(s, axis=-1, keepdims=True)
    var_s = jnp.mean((s - mu_s) ** 2, axis=-1, keepdims=True)
    s_n = (s - mu_s) * jax.lax.rsqrt(var_s + 1e-5) * o3_w0_ref[...] + o3_b0_ref[...]
    inv_rms1 = jax.lax.rsqrt(jnp.sum(v1 * v1, axis=-1, keepdims=True) / 64.0 + 1e-5)
    v1_n = v1 * inv_rms1 * w1rep_ref[...]
    inv_rms2 = jax.lax.rsqrt(jnp.sum(v2 * v2, axis=-1, keepdims=True) / 32.0 + 1e-5)
    v2_n = v2 * inv_rms2 * w2rep_ref[...]
    ne_out_ref[...] = jnp.concatenate([s_n, v1_n, v2_n], axis=-1)
    hpre = ns @ W1_ref[...] + b1_ref[...]
    h_ref[...] = hpre * jax.nn.sigmoid(hpre)


def _edge_kernel(hg_ref, neg_ref, rsh_ref, rbf_ref, fcut_ref,
                 W2e_ref, b2e_ref, rbf_We_ref, rbf_be_ref,
                 m0_ref, m1_ref, m2_ref, m3_ref, m4_ref):
    hg = hg_ref[...]
    fwe = (rbf_ref[...] @ rbf_We_ref[...] + rbf_be_ref[...]) * fcut_ref[...]
    foe = (hg @ W2e_ref[...] + b2e_ref[...]) * fwe
    msg_eq = neg_ref[...] * foe[:, :480] + rsh_ref[...] * foe[:, 480:960]
    m0_ref[...] = foe[:, 960:1088]
    m1_ref[...] = msg_eq[:, :128]
    m2_ref[...] = msg_eq[:, 128:256]
    m3_ref[...] = msg_eq[:, 256:384]
    bs = msg_eq.shape[0]
    m4_ref[...] = jnp.concatenate(
        [msg_eq[:, 384:480], jnp.zeros((bs, 32), jnp.float32)], axis=-1)


def _sc_phase(cid, tid, m_hbm, init_hbm, out_hbm, center_hbm,
              acc, bufs, idxs, sems, buf_t, idx_t):
    r0 = tid * ROWS_PER_TILE
    pltpu.sync_copy(init_hbm.at[pl.ds(cid * NPAD + r0, ROWS_PER_TILE)],
                    acc.at[pl.ds(r0, ROWS_PER_TILE)])
    plsc.subcore_barrier()

    ebase = cid * (EC // NCORES) + tid * EDGES_PER_TILE
    buf0, buf1 = bufs
    idx0, idx1 = idxs
    semm0, semi0, semm1, semi1 = sems

    def start(j, buf, idxbuf, semm, semi):
        e0 = ebase + j * CHUNK
        pltpu.async_copy(m_hbm.at[pl.ds(e0, CHUNK)], buf, semm)
        pltpu.async_copy(center_hbm.at[pl.ds(e0, CHUNK)], idxbuf, semi)

    def wait_scatter(buf, idxbuf, semm, semi):
        pltpu.make_async_copy(m_hbm.at[pl.ds(0, CHUNK)], buf, semm).wait()
        pltpu.make_async_copy(center_hbm.at[pl.ds(0, CHUNK)], idxbuf, semi).wait()
        pltpu.sync_copy(buf, acc.at[idxbuf], add=True)

    start(0, buf0, idx0, semm0, semi0)

    def body(i, carry):
        start(2 * i + 1, buf1, idx1, semm1, semi1)
        wait_scatter(buf0, idx0, semm0, semi0)

        @pl.when(2 * i + 2 < NCHUNK)
        def _():
            start(2 * i + 2, buf0, idx0, semm0, semi0)

        wait_scatter(buf1, idx1, semm1, semi1)
        return carry

    lax.fori_loop(0, NCHUNK // 2, body, 0)
    e0t = ebase + NCHUNK * CHUNK
    pltpu.sync_copy(m_hbm.at[pl.ds(e0t, TAIL)], buf_t)
    pltpu.sync_copy(center_hbm.at[pl.ds(e0t, TAIL)], idx_t)
    pltpu.sync_copy(buf_t, acc.at[idx_t], add=True)
    plsc.subcore_barrier()
    pltpu.sync_copy(acc.at[pl.ds(r0, ROWS_PER_TILE)],
                    out_hbm.at[pl.ds(cid * NPAD + r0, ROWS_PER_TILE)])
    plsc.subcore_barrier()


def _make_scatter_kernel():
    mesh = plsc.VectorSubcoreMesh(core_axis_name="c", subcore_axis_name="s")
    out_type = [jax.ShapeDtypeStruct((NCORES * NPAD, W), jnp.float32)
                for _ in range(NBLK)]
    scratch_types = [
        pltpu.VMEM_SHARED((NPAD, W), jnp.float32),
        pltpu.VMEM((CHUNK, W), jnp.float32),
        pltpu.VMEM((CHUNK, W), jnp.float32),
        pltpu.VMEM((CHUNK,), jnp.int32),
        pltpu.VMEM((CHUNK,), jnp.int32),
        pltpu.SemaphoreType.DMA,
        pltpu.SemaphoreType.DMA,
        pltpu.SemaphoreType.DMA,
        pltpu.SemaphoreType.DMA,
        pltpu.VMEM((TAIL, W), jnp.float32),
        pltpu.VMEM((TAIL,), jnp.int32),
    ]

    @functools.partial(pl.kernel, mesh=mesh, out_type=out_type,
                       scratch_types=scratch_types)
    def scatter_kernel(m0, m1, m2, m3, m4, center, i0, i1, i2, i3, i4,
                       o0, o1, o2, o3, o4, acc, buf0, buf1, idx0, idx1,
                       semm0, semi0, semm1, semi1, buf_t, idx_t):
        cid = lax.axis_index("c")
        tid = lax.axis_index("s")
        for m, i, o in ((m0, i0, o0), (m1, i1, o1), (m2, i2, o2),
                        (m3, i3, o3), (m4, i4, o4)):
            _sc_phase(cid, tid, m, i, o, center, acc,
                      (buf0, buf1), (idx0, idx1),
                      (semm0, semi0, semm1, semi1), buf_t, idx_t)

    return scatter_kernel


def kernel(node_invariant, node_equivariant, rbf, fcut, rsh, edge_index, ln_w, ln_b, o3_w0, o3_b0, o3_w1, o3_w2, W1, b1, W2, b2, rbf_W, rbf_b):
    w1rep = jnp.repeat(o3_w1, 3)
    w2rep = jnp.repeat(o3_w2, 5)
    mfull = jnp.asarray(_MFULL, dtype=jnp.int32)
    W2e = W2[:, mfull]
    b2e = b2[mfull]
    rbf_We = rbf_W[:, mfull]
    rbf_be = rbf_b[mfull]

    nb = 10
    bs = N // nb
    h, ne_norm = pl.pallas_call(
        _node_kernel,
        grid=(nb,),
        in_specs=[
            pl.BlockSpec((bs, 128), lambda i: (i, 0)),
            pl.BlockSpec((bs, 480), lambda i: (i, 0)),
            pl.BlockSpec((128,), lambda i: (0,)),
            pl.BlockSpec((128,), lambda i: (0,)),
            pl.BlockSpec((128,), lambda i: (0,)),
            pl.BlockSpec((128,), lambda i: (0,)),
            pl.BlockSpec((192,), lambda i: (0,)),
            pl.BlockSpec((160,), lambda i: (0,)),
            pl.BlockSpec((128, 128), lambda i: (0, 0)),
            pl.BlockSpec((128,), lambda i: (0,)),
        ],
        out_specs=[
            pl.BlockSpec((bs, 128), lambda i: (i, 0)),
            pl.BlockSpec((bs, 480), lambda i: (i, 0)),
        ],
        out_shape=[
            jax.ShapeDtypeStruct((N, 128), jnp.float32),
            jax.ShapeDtypeStruct((N, 480), jnp.float32),
        ],
    )(node_invariant, node_equivariant, ln_w, ln_b, o3_w0, o3_b0,
      w1rep, w2rep, W1, b1)

    center = edge_index[0]
    neigh = edge_index[1]

    eb = 160
    ebs = EC // eb
    scatter = _make_scatter_kernel()
    parts = [jnp.zeros((NCORES * NPAD, W), jnp.float32) for _ in range(NBLK)]
    for k in range(KCHUNKS):
        sl = slice(k * EC, (k + 1) * EC)
        neigh_k = neigh[sl]
        h_g = h[neigh_k]
        ne_g = ne_norm[neigh_k]
        msgs = pl.pallas_call(
            _edge_kernel,
            grid=(eb,),
            in_specs=[
                pl.BlockSpec((ebs, 128), lambda i: (i, 0)),
                pl.BlockSpec((ebs, 480), lambda i: (i, 0)),
                pl.BlockSpec((ebs, 480), lambda i: (i, 0)),
                pl.BlockSpec((ebs, 20), lambda i: (i, 0)),
                pl.BlockSpec((ebs, 1), lambda i: (i, 0)),
                pl.BlockSpec((128, 1088), lambda i: (0, 0)),
                pl.BlockSpec((1088,), lambda i: (0,)),
                pl.BlockSpec((20, 1088), lambda i: (0, 0)),
                pl.BlockSpec((1088,), lambda i: (0,)),
            ],
            out_specs=[
                pl.BlockSpec((ebs, W), lambda i: (i, 0)) for _ in range(NBLK)
            ],
            out_shape=[jax.ShapeDtypeStruct((EC, W), jnp.float32)
                       for _ in range(NBLK)],
        )(h_g, ne_g, rsh[sl], rbf[sl], fcut[sl], W2e, b2e, rbf_We, rbf_be)

        parts = list(scatter(*msgs, center[sl], *parts))

    o0, o1, o2, o3, o4 = parts
    new_inv = node_invariant + o0[:N] + o0[NPAD:NPAD + N]
    eqs = [o[:N] + o[NPAD:NPAD + N] for o in (o1, o2, o3, o4)]
    new_eq = node_equivariant + jnp.concatenate(
        [eqs[0], eqs[1], eqs[2], eqs[3][:, :96]], axis=1)
    return new_inv, new_eq
